# baseline (speedup 1.0000x reference)
import functools

import jax
import jax.numpy as jnp
from jax import lax
from jax.experimental import pallas as pl
from jax.experimental.pallas import tpu as pltpu
from jax.experimental.pallas import tpu_sc as plsc

DICT_SIZE = 1000000
EMBED_DIM = 64
BATCH = 16384
HIST = 20
TOTAL = BATCH * HIST

NUM_CORES = 2
NUM_SUBCORES = 16
NW = NUM_CORES * NUM_SUBCORES

PER_W = TOTAL // NW
CHUNK = 128
NCHUNK = PER_W // CHUNK
PADW = 128
NSLOT = 10
LOOKAHEAD = 5

WBLK = 2048
NGRID = (DICT_SIZE + WBLK - 1) // WBLK

_mesh = plsc.VectorSubcoreMesh(core_axis_name="c", subcore_axis_name="s")


def _format_body(wt_ref, out_ref):
    t = jnp.swapaxes(wt_ref[...], 0, 1)
    t3 = t.reshape(WBLK // 2, 2, EMBED_DIM)
    out_ref[...] = jnp.concatenate([t3[:, 0, :], t3[:, 1, :]], axis=1)


_table_format = pl.pallas_call(
    _format_body,
    grid=(NGRID,),
    in_specs=[pl.BlockSpec((EMBED_DIM, WBLK), lambda i: (0, i))],
    out_specs=pl.BlockSpec((WBLK // 2, PADW), lambda i: (i, 0)),
    out_shape=jax.ShapeDtypeStruct((DICT_SIZE // 2, PADW), jnp.float32),
)


@functools.partial(
    pl.kernel,
    out_type=jax.ShapeDtypeStruct((TOTAL, EMBED_DIM), jnp.float32),
    mesh=_mesh,
    scratch_types=[
        pltpu.VMEM((NCHUNK, CHUNK), jnp.int32),
        pltpu.VMEM((NSLOT, CHUNK, EMBED_DIM), jnp.float32),
        pltpu.SemaphoreType.DMA,
        [pltpu.SemaphoreType.DMA] * NSLOT,
        [pltpu.SemaphoreType.DMA] * NSLOT,
    ],
    compiler_params=pltpu.CompilerParams(use_tc_tiling_on_sc=False),
)
def _emb_lookup(idx_hbm, table_hbm, out_hbm, idx_v, rows_v, isem, gsems, wsems):
    wid = lax.axis_index("s") * NUM_CORES + lax.axis_index("c")
    base = wid * PER_W

    pltpu.async_copy(idx_hbm.at[pl.ds(wid * NCHUNK, NCHUNK)], idx_v, isem).wait()

    for b in range(LOOKAHEAD):
        pltpu.async_copy(table_hbm.at[idx_v.at[b]], rows_v.at[b], gsems[b])

    @pl.loop(0, NCHUNK, step=NSLOT)
    def _group(g):
        for b in range(NSLOT):
            j = g + b
            jn = j + LOOKAHEAD
            bn = (b + LOOKAHEAD) % NSLOT

            @pl.when(jn < NCHUNK)
            def _():
                @pl.when(jn >= NSLOT)
                def _():
                    pltpu.make_async_copy(
                        rows_v.at[bn],
                        out_hbm.at[pl.ds(base, CHUNK)],
                        wsems[bn],
                    ).wait()

                pltpu.async_copy(table_hbm.at[idx_v.at[jn]], rows_v.at[bn], gsems[bn])

            pltpu.make_async_copy(
                table_hbm.at[idx_v.at[b]], rows_v.at[b], gsems[b]
            ).wait()
            pltpu.async_copy(
                rows_v.at[b],
                out_hbm.at[pl.ds(base + j * CHUNK, CHUNK)],
                wsems[b],
            )

    for b in range(NSLOT):
        pltpu.make_async_copy(
            rows_v.at[b], out_hbm.at[pl.ds(base, CHUNK)], wsems[b]
        ).wait()


def kernel(x, weight):
    idx2d = x.astype(jnp.int32).reshape(TOTAL // CHUNK, CHUNK)
    table = _table_format(weight.T).reshape(DICT_SIZE, EMBED_DIM)
    out = _emb_lookup(idx2d, table)
    return out.reshape(BATCH, HIST, EMBED_DIM)


# speedup vs baseline: 1.2457x; 1.2457x over previous
import functools

import jax
import jax.numpy as jnp
from jax import lax
from jax.experimental import pallas as pl
from jax.experimental.pallas import tpu as pltpu
from jax.experimental.pallas import tpu_sc as plsc

DICT_SIZE = 1000000
EMBED_DIM = 64
BATCH = 16384
HIST = 20
TOTAL = BATCH * HIST

NUM_CORES = 2
NUM_SUBCORES = 16
NW = NUM_CORES * NUM_SUBCORES

PER_W = TOTAL // NW
CHUNK = 128
NCHUNK = PER_W // CHUNK
PADW = 128
NSLOT = 5
LOOKAHEAD = 2

WBLK = 4096
NGRID = (DICT_SIZE + WBLK - 1) // WBLK

_mesh = plsc.VectorSubcoreMesh(core_axis_name="c", subcore_axis_name="s")


def _format_body(wt_ref, out_ref):
    out_ref[:, 0:EMBED_DIM] = jnp.swapaxes(wt_ref[...], 0, 1)


_table_format = pl.pallas_call(
    _format_body,
    grid=(NGRID,),
    in_specs=[pl.BlockSpec((EMBED_DIM, WBLK), lambda i: (0, i))],
    out_specs=pl.BlockSpec((WBLK, PADW), lambda i: (i, 0)),
    out_shape=jax.ShapeDtypeStruct((DICT_SIZE, PADW), jnp.float32),
)


@functools.partial(
    pl.kernel,
    out_type=jax.ShapeDtypeStruct((TOTAL, EMBED_DIM), jnp.float32),
    mesh=_mesh,
    scratch_types=[
        pltpu.VMEM((NCHUNK, CHUNK), jnp.int32),
        pltpu.VMEM((NSLOT, CHUNK, PADW), jnp.float32),
        pltpu.SemaphoreType.DMA,
        [pltpu.SemaphoreType.DMA] * NSLOT,
        [pltpu.SemaphoreType.DMA] * NSLOT,
    ],
    compiler_params=pltpu.CompilerParams(use_tc_tiling_on_sc=False),
)
def _emb_lookup(idx_hbm, table_hbm, out_hbm, idx_v, rows_v, isem, gsems, wsems):
    wid = lax.axis_index("s") * NUM_CORES + lax.axis_index("c")
    base = wid * PER_W

    pltpu.async_copy(idx_hbm.at[pl.ds(wid * NCHUNK, NCHUNK)], idx_v, isem).wait()

    for b in range(LOOKAHEAD):
        pltpu.async_copy(table_hbm.at[idx_v.at[b]], rows_v.at[b], gsems[b])

    @pl.loop(0, NCHUNK, step=NSLOT)
    def _group(g):
        for b in range(NSLOT):
            j = g + b
            jn = j + LOOKAHEAD
            bn = (b + LOOKAHEAD) % NSLOT

            @pl.when(jn < NCHUNK)
            def _():
                @pl.when(jn >= NSLOT)
                def _():
                    pltpu.make_async_copy(
                        rows_v.at[bn, :, pl.ds(0, EMBED_DIM)],
                        out_hbm.at[pl.ds(base, CHUNK)],
                        wsems[bn],
                    ).wait()

                pltpu.async_copy(table_hbm.at[idx_v.at[jn]], rows_v.at[bn], gsems[bn])

            pltpu.make_async_copy(
                table_hbm.at[idx_v.at[b]], rows_v.at[b], gsems[b]
            ).wait()
            pltpu.async_copy(
                rows_v.at[b, :, pl.ds(0, EMBED_DIM)],
                out_hbm.at[pl.ds(base + j * CHUNK, CHUNK)],
                wsems[b],
            )

    for b in range(NSLOT):
        pltpu.make_async_copy(
            rows_v.at[b, :, pl.ds(0, EMBED_DIM)],
            out_hbm.at[pl.ds(base, CHUNK)],
            wsems[b],
        ).wait()


def kernel(x, weight):
    idx2d = x.astype(jnp.int32).reshape(TOTAL // CHUNK, CHUNK)
    table = _table_format(weight.T)
    out = _emb_lookup(idx2d, table)
    return out.reshape(BATCH, HIST, EMBED_DIM)


# speedup vs baseline: 1.3977x; 1.1220x over previous
import functools

import jax
import jax.numpy as jnp
from jax import lax
from jax.experimental import pallas as pl
from jax.experimental.pallas import tpu as pltpu
from jax.experimental.pallas import tpu_sc as plsc

DICT_SIZE = 1000000
EMBED_DIM = 64
BATCH = 16384
HIST = 20
TOTAL = BATCH * HIST

NUM_CORES = 2
NUM_SUBCORES = 16
NW = NUM_CORES * NUM_SUBCORES

PER_W = TOTAL // NW
CHUNK = 128
NCHUNK = PER_W // CHUNK
PADW = 128
NSLOT = 5
LOOKAHEAD = 2

WBLK = 8192
NGRID = (DICT_SIZE + WBLK - 1) // WBLK

_mesh = plsc.VectorSubcoreMesh(core_axis_name="c", subcore_axis_name="s")


def _format_body(wt_ref, out_ref):
    out_ref[:, 0:EMBED_DIM] = jnp.swapaxes(wt_ref[...], 0, 1)


_table_format = pl.pallas_call(
    _format_body,
    grid=(NGRID,),
    in_specs=[pl.BlockSpec((EMBED_DIM, WBLK), lambda i: (0, i))],
    out_specs=pl.BlockSpec((WBLK, PADW), lambda i: (i, 0)),
    out_shape=jax.ShapeDtypeStruct((DICT_SIZE, PADW), jnp.float32),
)


@functools.partial(
    pl.kernel,
    out_type=jax.ShapeDtypeStruct((TOTAL, EMBED_DIM), jnp.float32),
    mesh=_mesh,
    scratch_types=[
        pltpu.VMEM((NCHUNK, CHUNK), jnp.int32),
        pltpu.VMEM((NSLOT, CHUNK, PADW), jnp.float32),
        pltpu.SemaphoreType.DMA,
        [pltpu.SemaphoreType.DMA] * NSLOT,
        [pltpu.SemaphoreType.DMA] * NSLOT,
    ],
    compiler_params=pltpu.CompilerParams(use_tc_tiling_on_sc=False),
)
def _emb_lookup(idx_hbm, table_hbm, out_hbm, idx_v, rows_v, isem, gsems, wsems):
    wid = lax.axis_index("s") * NUM_CORES + lax.axis_index("c")
    base = wid * PER_W

    pltpu.async_copy(idx_hbm.at[pl.ds(wid * NCHUNK, NCHUNK)], idx_v, isem).wait()

    for b in range(LOOKAHEAD):
        pltpu.async_copy(table_hbm.at[idx_v.at[b]], rows_v.at[b], gsems[b])

    @pl.loop(0, NCHUNK, step=NSLOT)
    def _group(g):
        for b in range(NSLOT):
            j = g + b
            jn = j + LOOKAHEAD
            bn = (b + LOOKAHEAD) % NSLOT

            @pl.when(jn < NCHUNK)
            def _():
                @pl.when(jn >= NSLOT)
                def _():
                    pltpu.make_async_copy(
                        rows_v.at[bn, :, pl.ds(0, EMBED_DIM)],
                        out_hbm.at[pl.ds(base, CHUNK)],
                        wsems[bn],
                    ).wait()

                pltpu.async_copy(table_hbm.at[idx_v.at[jn]], rows_v.at[bn], gsems[bn])

            pltpu.make_async_copy(
                table_hbm.at[idx_v.at[b]], rows_v.at[b], gsems[b]
            ).wait()
            pltpu.async_copy(
                rows_v.at[b, :, pl.ds(0, EMBED_DIM)],
                out_hbm.at[pl.ds(base + j * CHUNK, CHUNK)],
                wsems[b],
            )

    for b in range(NSLOT):
        pltpu.make_async_copy(
            rows_v.at[b, :, pl.ds(0, EMBED_DIM)],
            out_hbm.at[pl.ds(base, CHUNK)],
            wsems[b],
        ).wait()


def kernel(x, weight):
    idx2d = x.astype(jnp.int32).reshape(TOTAL // CHUNK, CHUNK)
    table = _table_format(weight.T)
    out = _emb_lookup(idx2d, table)
    return out.reshape(BATCH, HIST, EMBED_DIM)


# speedup vs baseline: 1.4493x; 1.0370x over previous
import functools

import jax
import jax.numpy as jnp
from jax import lax
from jax.experimental import pallas as pl
from jax.experimental.pallas import tpu as pltpu
from jax.experimental.pallas import tpu_sc as plsc

DICT_SIZE = 1000000
EMBED_DIM = 64
BATCH = 16384
HIST = 20
TOTAL = BATCH * HIST

NUM_CORES = 2
NUM_SUBCORES = 16
NW = NUM_CORES * NUM_SUBCORES

PER_W = TOTAL // NW
CHUNK = 128
NCHUNK = PER_W // CHUNK
PADW = 128
NSLOT = 5
LOOKAHEAD = 2

WBLK = 16384
NGRID = (DICT_SIZE + WBLK - 1) // WBLK

_mesh = plsc.VectorSubcoreMesh(core_axis_name="c", subcore_axis_name="s")


def _format_body(wt_ref, out_ref):
    out_ref[:, 0:EMBED_DIM] = jnp.swapaxes(wt_ref[...], 0, 1)


_table_format = pl.pallas_call(
    _format_body,
    grid=(NGRID,),
    in_specs=[pl.BlockSpec((EMBED_DIM, WBLK), lambda i: (0, i))],
    out_specs=pl.BlockSpec((WBLK, PADW), lambda i: (i, 0)),
    out_shape=jax.ShapeDtypeStruct((DICT_SIZE, PADW), jnp.float32),
)


@functools.partial(
    pl.kernel,
    out_type=jax.ShapeDtypeStruct((TOTAL, EMBED_DIM), jnp.float32),
    mesh=_mesh,
    scratch_types=[
        pltpu.VMEM((NCHUNK, CHUNK), jnp.int32),
        pltpu.VMEM((NSLOT, CHUNK, PADW), jnp.float32),
        pltpu.SemaphoreType.DMA,
        [pltpu.SemaphoreType.DMA] * NSLOT,
        [pltpu.SemaphoreType.DMA] * NSLOT,
    ],
    compiler_params=pltpu.CompilerParams(use_tc_tiling_on_sc=False),
)
def _emb_lookup(idx_hbm, table_hbm, out_hbm, idx_v, rows_v, isem, gsems, wsems):
    wid = lax.axis_index("s") * NUM_CORES + lax.axis_index("c")
    base = wid * PER_W

    pltpu.async_copy(idx_hbm.at[pl.ds(wid * NCHUNK, NCHUNK)], idx_v, isem).wait()

    for b in range(LOOKAHEAD):
        pltpu.async_copy(table_hbm.at[idx_v.at[b]], rows_v.at[b], gsems[b])

    @pl.loop(0, NCHUNK, step=NSLOT)
    def _group(g):
        for b in range(NSLOT):
            j = g + b
            jn = j + LOOKAHEAD
            bn = (b + LOOKAHEAD) % NSLOT

            @pl.when(jn < NCHUNK)
            def _():
                @pl.when(jn >= NSLOT)
                def _():
                    pltpu.make_async_copy(
                        rows_v.at[bn, :, pl.ds(0, EMBED_DIM)],
                        out_hbm.at[pl.ds(base, CHUNK)],
                        wsems[bn],
                    ).wait()

                pltpu.async_copy(table_hbm.at[idx_v.at[jn]], rows_v.at[bn], gsems[bn])

            pltpu.make_async_copy(
                table_hbm.at[idx_v.at[b]], rows_v.at[b], gsems[b]
            ).wait()
            pltpu.async_copy(
                rows_v.at[b, :, pl.ds(0, EMBED_DIM)],
                out_hbm.at[pl.ds(base + j * CHUNK, CHUNK)],
                wsems[b],
            )

    for b in range(NSLOT):
        pltpu.make_async_copy(
            rows_v.at[b, :, pl.ds(0, EMBED_DIM)],
            out_hbm.at[pl.ds(base, CHUNK)],
            wsems[b],
        ).wait()


def kernel(x, weight):
    idx2d = x.astype(jnp.int32).reshape(TOTAL // CHUNK, CHUNK)
    table = _table_format(weight.T)
    out = _emb_lookup(idx2d, table)
    return out.reshape(BATCH, HIST, EMBED_DIM)
